# baseline (speedup 1.0000x reference)
import jax
import jax.numpy as jnp
from jax.experimental import pallas as pl
from jax.experimental.pallas import tpu as pltpu

_N0, _N1 = 2048, 4096
_IN, _HID, _OUT = 128, 32, 32
_KAUG = 256
_BM = 256
_NB = _N0 // _BM


def _fused_kernel(x0_ref, x1_ref, a_ref, b_ref,
                  w0_ref, b0_ref, w1_ref, b1_ref,
                  w00_ref, w10_ref, wout_ref, bout_ref,
                  out_ref, x0s_ref, x1s_ref, g0_ref, g1_ref, acc_ref):
    i = pl.program_id(0)

    @pl.when(i == 0)
    def _setup():
        lane0 = jax.lax.broadcasted_iota(jnp.int32, (_N0, _KAUG - _IN), 1)
        x0s_ref[...] = jnp.concatenate(
            [x0_ref[...], jnp.where(lane0 == 0, 1.0, 0.0)],
            axis=1).astype(jnp.bfloat16)
        lane1 = jax.lax.broadcasted_iota(jnp.int32, (_N1, _KAUG - _IN), 1)
        x1s_ref[...] = jnp.concatenate(
            [x1_ref[...], jnp.where(lane1 == 0, 1.0, 0.0)],
            axis=1).astype(jnp.bfloat16)

        row = jax.lax.broadcasted_iota(jnp.int32, (_KAUG - _IN, _HID), 0)
        b0row = jnp.where(row == 0, b0_ref[...], 0.0)
        w0aug = jnp.concatenate([w0_ref[...], b0row], axis=0)
        g0_ref[...] = jnp.dot(w0aug, w00_ref[...],
                              preferred_element_type=jnp.float32
                              ).astype(jnp.bfloat16)
        b1row = jnp.where(row == 0, b1_ref[...], 0.0)
        w1aug = jnp.concatenate([w1_ref[...], b1row], axis=0)
        g1_ref[...] = jnp.dot(w1aug, w10_ref[...],
                              preferred_element_type=jnp.float32
                              ).astype(jnp.bfloat16)
        acc_ref[...] = jnp.zeros_like(acc_ref)

    p_a = jnp.dot(a_ref[...].astype(jnp.bfloat16), x0s_ref[...],
                  preferred_element_type=jnp.float32)
    p_b = jnp.dot(b_ref[...].astype(jnp.bfloat16), x1s_ref[...],
                  preferred_element_type=jnp.float32)
    m = (jnp.dot(p_a.astype(jnp.bfloat16), g0_ref[...],
                 preferred_element_type=jnp.float32)
         + jnp.dot(p_b.astype(jnp.bfloat16), g1_ref[...],
                   preferred_element_type=jnp.float32))
    acc_ref[...] += jnp.sum(jax.nn.sigmoid(m), axis=0, keepdims=True)

    @pl.when(i == _NB - 1)
    def _epilogue():
        mean = acc_ref[...] * (1.0 / _N0)
        out_ref[...] = jnp.dot(mean, wout_ref[...],
                               preferred_element_type=jnp.float32) + bout_ref[...]


def kernel(x_0, x_1, x_2, incidence_1, incidence_1_norm, incidence_2,
           incidence_2_norm, adjacency_up_0_norm, adjacency_up_1_norm,
           adjacency_down_1_norm, adjacency_down_2_norm,
           W0_in, b0_in, W1_in, b1_in, W2_in, b2_in,
           w_0_to_0, w_1_to_0, w_0_to_1, w_1_to_1, w_2_to_1, w_1_to_2,
           w_2_to_2, W0_out, b0_out, W1_out, b1_out, W2_out, b2_out):
    const = lambda i: (0, 0)
    return pl.pallas_call(
        _fused_kernel,
        grid=(_NB,),
        in_specs=[
            pl.BlockSpec((_N0, _IN), const),
            pl.BlockSpec((_N1, _IN), const),
            pl.BlockSpec((_BM, _N0), lambda i: (i, 0)),
            pl.BlockSpec((_BM, _N1), lambda i: (i, 0)),
            pl.BlockSpec((_IN, _HID), const),
            pl.BlockSpec((1, _HID), const),
            pl.BlockSpec((_IN, _HID), const),
            pl.BlockSpec((1, _HID), const),
            pl.BlockSpec((_HID, _HID), const),
            pl.BlockSpec((_HID, _HID), const),
            pl.BlockSpec((_HID, _OUT), const),
            pl.BlockSpec((1, _OUT), const),
        ],
        out_specs=pl.BlockSpec((1, _OUT), const),
        out_shape=jax.ShapeDtypeStruct((1, _OUT), jnp.float32),
        scratch_shapes=[
            pltpu.VMEM((_N0, _KAUG), jnp.bfloat16),
            pltpu.VMEM((_N1, _KAUG), jnp.bfloat16),
            pltpu.VMEM((_KAUG, _HID), jnp.bfloat16),
            pltpu.VMEM((_KAUG, _HID), jnp.bfloat16),
            pltpu.VMEM((1, _OUT), jnp.float32),
        ],
    )(x_0, x_1, adjacency_up_0_norm, incidence_1_norm,
      W0_in, b0_in.reshape(1, _HID), W1_in, b1_in.reshape(1, _HID),
      w_0_to_0, w_1_to_0, W0_out, b0_out.reshape(1, _OUT))


# speedup vs baseline: 1.0324x; 1.0324x over previous
import jax
import jax.numpy as jnp
from jax.experimental import pallas as pl
from jax.experimental.pallas import tpu as pltpu

_N0, _N1 = 2048, 4096
_IN, _HID, _OUT = 128, 32, 32
_KAUG = 256
_BM = 512
_NB = _N0 // _BM


def _fused_kernel(x0_ref, x1_ref, a_ref, b_ref,
                  w0_ref, b0_ref, w1_ref, b1_ref,
                  w00_ref, w10_ref, wout_ref, bout_ref,
                  out_ref, x0s_ref, x1s_ref, g0_ref, g1_ref, acc_ref):
    i = pl.program_id(0)

    @pl.when(i == 0)
    def _setup():
        lane0 = jax.lax.broadcasted_iota(jnp.int32, (_N0, _KAUG - _IN), 1)
        x0s_ref[...] = jnp.concatenate(
            [x0_ref[...], jnp.where(lane0 == 0, 1.0, 0.0)],
            axis=1).astype(jnp.bfloat16)
        lane1 = jax.lax.broadcasted_iota(jnp.int32, (_N1, _KAUG - _IN), 1)
        x1s_ref[...] = jnp.concatenate(
            [x1_ref[...], jnp.where(lane1 == 0, 1.0, 0.0)],
            axis=1).astype(jnp.bfloat16)

        row = jax.lax.broadcasted_iota(jnp.int32, (_KAUG - _IN, _HID), 0)
        b0row = jnp.where(row == 0, b0_ref[...], 0.0)
        w0aug = jnp.concatenate([w0_ref[...], b0row], axis=0)
        g0_ref[...] = jnp.dot(w0aug, w00_ref[...],
                              preferred_element_type=jnp.float32
                              ).astype(jnp.bfloat16)
        b1row = jnp.where(row == 0, b1_ref[...], 0.0)
        w1aug = jnp.concatenate([w1_ref[...], b1row], axis=0)
        g1_ref[...] = jnp.dot(w1aug, w10_ref[...],
                              preferred_element_type=jnp.float32
                              ).astype(jnp.bfloat16)
        acc_ref[...] = jnp.zeros_like(acc_ref)

    p_a = jnp.dot(a_ref[...].astype(jnp.bfloat16), x0s_ref[...],
                  preferred_element_type=jnp.float32)
    p_b = jnp.dot(b_ref[...].astype(jnp.bfloat16), x1s_ref[...],
                  preferred_element_type=jnp.float32)
    m = (jnp.dot(p_a.astype(jnp.bfloat16), g0_ref[...],
                 preferred_element_type=jnp.float32)
         + jnp.dot(p_b.astype(jnp.bfloat16), g1_ref[...],
                   preferred_element_type=jnp.float32))
    acc_ref[...] += jnp.sum(jax.nn.sigmoid(m), axis=0, keepdims=True)

    @pl.when(i == _NB - 1)
    def _epilogue():
        mean = acc_ref[...] * (1.0 / _N0)
        out_ref[...] = jnp.dot(mean, wout_ref[...],
                               preferred_element_type=jnp.float32) + bout_ref[...]


def kernel(x_0, x_1, x_2, incidence_1, incidence_1_norm, incidence_2,
           incidence_2_norm, adjacency_up_0_norm, adjacency_up_1_norm,
           adjacency_down_1_norm, adjacency_down_2_norm,
           W0_in, b0_in, W1_in, b1_in, W2_in, b2_in,
           w_0_to_0, w_1_to_0, w_0_to_1, w_1_to_1, w_2_to_1, w_1_to_2,
           w_2_to_2, W0_out, b0_out, W1_out, b1_out, W2_out, b2_out):
    const = lambda i: (0, 0)
    return pl.pallas_call(
        _fused_kernel,
        grid=(_NB,),
        in_specs=[
            pl.BlockSpec((_N0, _IN), const),
            pl.BlockSpec((_N1, _IN), const),
            pl.BlockSpec((_BM, _N0), lambda i: (i, 0)),
            pl.BlockSpec((_BM, _N1), lambda i: (i, 0)),
            pl.BlockSpec((_IN, _HID), const),
            pl.BlockSpec((1, _HID), const),
            pl.BlockSpec((_IN, _HID), const),
            pl.BlockSpec((1, _HID), const),
            pl.BlockSpec((_HID, _HID), const),
            pl.BlockSpec((_HID, _HID), const),
            pl.BlockSpec((_HID, _OUT), const),
            pl.BlockSpec((1, _OUT), const),
        ],
        out_specs=pl.BlockSpec((1, _OUT), const),
        out_shape=jax.ShapeDtypeStruct((1, _OUT), jnp.float32),
        scratch_shapes=[
            pltpu.VMEM((_N0, _KAUG), jnp.bfloat16),
            pltpu.VMEM((_N1, _KAUG), jnp.bfloat16),
            pltpu.VMEM((_KAUG, _HID), jnp.bfloat16),
            pltpu.VMEM((_KAUG, _HID), jnp.bfloat16),
            pltpu.VMEM((1, _OUT), jnp.float32),
        ],
    )(x_0, x_1, adjacency_up_0_norm, incidence_1_norm,
      W0_in, b0_in.reshape(1, _HID), W1_in, b1_in.reshape(1, _HID),
      w_0_to_0, w_1_to_0, W0_out, b0_out.reshape(1, _OUT))
